# baseline (speedup 1.0000x reference)
import functools

import jax
import jax.numpy as jnp
from jax import lax
from jax.experimental import pallas as pl
from jax.experimental.pallas import tpu as pltpu
from jax.experimental.pallas import tpu_sc as plsc

B = 1024
T = 20
DEPTH = 1000
ROW = DEPTH * T

NUM_CORES = 2
NUM_SUBCORES = 16
NW = NUM_CORES * NUM_SUBCORES
ROWS_PER_W = B // NW


def _sc_one_hot(x_hbm, zsrc_hbm, out_hbm, x_v, zbuf):
    wid = lax.axis_index("s") * NUM_CORES + lax.axis_index("c")
    base = wid * ROWS_PER_W

    pltpu.sync_copy(x_hbm.at[pl.ds(base, ROWS_PER_W)], x_v)
    pltpu.sync_copy(zsrc_hbm, zbuf)

    iota = lax.iota(jnp.int32, 16)
    one_f = jnp.full((16,), 1.0, jnp.float32)
    zero_f = jnp.zeros((16,), jnp.float32)

    for i in range(ROWS_PER_W):
        xa = x_v[i, pl.ds(0, 16)]
        xb = x_v[i, pl.ds(4, 16)]
        ya = xa * T + iota
        yb = xb * T + (iota + 4)
        plsc.store_scatter(zbuf, [ya], one_f)
        plsc.store_scatter(zbuf, [yb], one_f)
        pltpu.sync_copy(zbuf, out_hbm.at[base + i])
        plsc.store_scatter(zbuf, [ya], zero_f)
        plsc.store_scatter(zbuf, [yb], zero_f)


@jax.jit
def _one_hot(x):
    zsrc = jnp.zeros((ROW,), jnp.float32)
    run = functools.partial(
        pl.kernel,
        out_type=jax.ShapeDtypeStruct((B, ROW), jnp.float32),
        mesh=plsc.VectorSubcoreMesh(core_axis_name="c", subcore_axis_name="s"),
        scratch_types=[
            pltpu.VMEM((ROWS_PER_W, T), jnp.int32),
            pltpu.VMEM((ROW,), jnp.float32),
        ],
        compiler_params=pltpu.CompilerParams(needs_layout_passes=False),
    )(_sc_one_hot)
    out_flat = run(x, zsrc)
    return out_flat.reshape(B, DEPTH, T)


def kernel(X_in, ones):
    del ones
    return _one_hot(X_in.astype(jnp.int32))
